# Initial kernel scaffold; baseline (speedup 1.0000x reference)
import jax
import jax.numpy as jnp
from jax.experimental import pallas as pl


def kernel(mu, t, context, edge_index, mp_params, edge_w1, edge_b1, edge_w2, edge_b2):
    raise NotImplementedError("write your pallas kernel here")


# speedup vs baseline: 1.0201x; 1.0201x over previous
import functools

import jax
import jax.numpy as jnp
from jax import lax
from jax.experimental import pallas as pl
from jax.experimental.pallas import tpu as pltpu


def _silu(x):
    return x / (1.0 + jnp.exp(-x))


def _fused_proj_kernel(h_ref, w_ref, b_ref, out_ref, *, act):
    x = h_ref[...]
    o = jax.lax.dot_general(x, w_ref[...], (((1,), (0,)), ((), ())),
                            preferred_element_type=jnp.float32)
    o = o + b_ref[...]
    if act:
        o = _silu(o)
    out_ref[...] = o


def _proj(h, w, b, act=False):
    R, K = h.shape
    F = w.shape[1]
    return pl.pallas_call(
        functools.partial(_fused_proj_kernel, act=act),
        out_shape=jax.ShapeDtypeStruct((R, F), jnp.float32),
    )(h, w, b)


def kernel(mu, t, context, edge_index, mp_params, edge_w1, edge_b1, edge_w2, edge_b2):
    B, N = mu.shape
    E = edge_index.shape[1]
    H = edge_w2.shape[0]

    t_exp = jnp.broadcast_to(t, (B, N))
    base = jnp.stack([mu, t_exp], axis=-1)
    h = jnp.concatenate([base, context], axis=-1).reshape(B * N, -1)

    src = edge_index[0]
    dst = edge_index[1]
    offsets = (jnp.arange(B) * N).astype(src.dtype)
    src_b = (src[None, :] + offsets[:, None]).reshape(-1)
    dst_b = (dst[None, :] + offsets[:, None]).reshape(-1)

    for (Wm, bm, Wn, bn) in mp_params:
        in_dim = h.shape[1]
        P = _proj(h, Wm[:in_dim], bm)
        Q = _proj(h, Wm[in_dim:], jnp.zeros_like(bm))
        m = _silu(P[src_b] + Q[dst_b])
        agg = jax.ops.segment_sum(m, dst_b, num_segments=B * N)
        h = _silu(_proj(h, Wn[:in_dim], bn) + _proj(agg, Wn[in_dim:], jnp.zeros_like(bn)))

    U = _proj(h, edge_w1[:H], edge_b1)
    V = _proj(h, edge_w1[H:], jnp.zeros_like(edge_b1))
    mid = _silu(U[src_b] + V[dst_b])
    logits = (mid @ edge_w2 + edge_b2).squeeze(-1)
    rates = jax.nn.softplus(logits).reshape(B, E)

    rm = jnp.zeros((B, N, N), jnp.float32)
    rm = rm.at[:, src, dst].set(rates)
    diag = jnp.arange(N)
    row_sum = rm.sum(axis=-1)
    rm = rm.at[:, diag, diag].set(-row_sum)
    return rm


# speedup vs baseline: 1.8617x; 1.8251x over previous
import functools

import jax
import jax.numpy as jnp
from jax import lax
from jax.experimental import pallas as pl
from jax.experimental.pallas import tpu as pltpu
from jax.experimental.pallas import tpu_sc as plsc

B, N, E, H = 4, 2048, 65536, 64
BN = B * N
BE = B * E
NC, NS, L = 2, 16, 16
ROWS_PER_CORE = BN // NC
CH = 512
CHUNKS = BE // (NC * NS * CH)
IDX_W = 128

_mesh = plsc.VectorSubcoreMesh(core_axis_name="c", subcore_axis_name="s")


def _silu(x):
    return x / (1.0 + jnp.exp(-x))


def _silu_vec(x):
    return x * (1.0 / (1.0 + jnp.exp(-x)))


def _proj2_body(h_ref, wp_ref, bp_ref, wq_ref, p_ref, q_ref):
    x = h_ref[...]
    p_ref[...] = lax.dot_general(x, wp_ref[...], (((1,), (0,)), ((), ())),
                                 preferred_element_type=jnp.float32) + bp_ref[...]
    q_ref[...] = lax.dot_general(x, wq_ref[...], (((1,), (0,)), ((), ())),
                                 preferred_element_type=jnp.float32)


def _proj2(h, wp, bp, wq):
    return pl.pallas_call(
        _proj2_body,
        out_shape=(jax.ShapeDtypeStruct((BN, H), jnp.float32),
                   jax.ShapeDtypeStruct((BN, H), jnp.float32)),
    )(h, wp, bp, wq)


def _update_body(h_ref, a_ref, wh_ref, b_ref, wa_ref, o_ref):
    o = lax.dot_general(h_ref[...], wh_ref[...], (((1,), (0,)), ((), ())),
                        preferred_element_type=jnp.float32)
    o = o + lax.dot_general(a_ref[...], wa_ref[...], (((1,), (0,)), ((), ())),
                            preferred_element_type=jnp.float32)
    o_ref[...] = _silu(o + b_ref[...])


def _update(h, agg, wh, b, wa):
    return pl.pallas_call(
        _update_body,
        out_shape=jax.ShapeDtypeStruct((BN, H), jnp.float32),
    )(h, agg, wh, b, wa)


def _edge_agg_body(p_hbm, q_hbm, srcb_hbm, dstb_hbm, out_hbm,
                   idxs, idxd, idxw, rows_p, rows_q, agg_sh, semp, semq):
    c = lax.axis_index("c")
    s = lax.axis_index("s")
    zero16 = jnp.zeros((L,), jnp.float32)
    my_sh0 = s * (ROWS_PER_CORE // NS)

    @pl.loop(0, ROWS_PER_CORE // NS)
    def _z(r):
        for k in range(H // L):
            rows_p[r, pl.ds(k * L, L)] = zero16
    pltpu.sync_copy(rows_p.at[pl.ds(0, ROWS_PER_CORE // NS)],
                    agg_sh.at[pl.ds(my_sh0, ROWS_PER_CORE // NS)])
    plsc.subcore_barrier()

    base_row = (c * NS + s) * (CHUNKS * CH // IDX_W)
    row_off = c * ROWS_PER_CORE

    @pl.loop(0, CHUNKS)
    def _chunk(g):
        row0 = base_row + g * (CH // IDX_W)
        pltpu.sync_copy(srcb_hbm.at[pl.ds(row0, CH // IDX_W)], idxs)
        pltpu.sync_copy(dstb_hbm.at[pl.ds(row0, CH // IDX_W)], idxd)
        for j in range(CH // IDX_W):
            for k in range(IDX_W // L):
                sl = pl.ds(k * L, L)
                idxw[j, sl] = idxd[j, sl] - row_off
        descs = []
        for j in range(CH // IDX_W):
            descs.append(pltpu.async_copy(
                p_hbm.at[idxs.at[j]], rows_p.at[pl.ds(j * IDX_W, IDX_W)], semp))
            descs.append(pltpu.async_copy(
                q_hbm.at[idxd.at[j]], rows_q.at[pl.ds(j * IDX_W, IDX_W)], semq))
        for d in descs:
            d.wait()

        @pl.loop(0, CH)
        def _compute(r):
            for k in range(H // L):
                sl = pl.ds(k * L, L)
                a = rows_p[r, sl] + rows_q[r, sl]
                rows_p[r, sl] = _silu_vec(a)

        for j in range(CH // IDX_W):
            pltpu.sync_copy(rows_p.at[pl.ds(j * IDX_W, IDX_W)],
                            agg_sh.at[idxw.at[j]], add=True)

    plsc.subcore_barrier()
    pltpu.sync_copy(agg_sh.at[pl.ds(my_sh0, ROWS_PER_CORE // NS)],
                    out_hbm.at[pl.ds(row_off + my_sh0, ROWS_PER_CORE // NS)])


_edge_agg = pl.kernel(
    _edge_agg_body,
    out_type=jax.ShapeDtypeStruct((BN, H), jnp.float32),
    mesh=_mesh,
    compiler_params=pltpu.CompilerParams(use_tc_tiling_on_sc=False),
    scratch_types=[
        pltpu.VMEM((CH // IDX_W, IDX_W), jnp.int32),
        pltpu.VMEM((CH // IDX_W, IDX_W), jnp.int32),
        pltpu.VMEM((CH // IDX_W, IDX_W), jnp.int32),
        pltpu.VMEM((CH, H), jnp.float32),
        pltpu.VMEM((CH, H), jnp.float32),
        pltpu.VMEM_SHARED((ROWS_PER_CORE, H), jnp.float32),
        pltpu.SemaphoreType.DMA,
        pltpu.SemaphoreType.DMA,
    ],
)


def kernel(mu, t, context, edge_index, mp_params, edge_w1, edge_b1, edge_w2, edge_b2):
    t_exp = jnp.broadcast_to(t, (B, N))
    base = jnp.stack([mu, t_exp], axis=-1)
    h = jnp.concatenate([base, context], axis=-1).reshape(BN, -1)

    src = edge_index[0]
    dst = edge_index[1]
    offsets = (jnp.arange(B) * N).astype(src.dtype)
    src_b = (src[None, :] + offsets[:, None]).reshape(-1).astype(jnp.int32)
    dst_b = (dst[None, :] + offsets[:, None]).reshape(-1).astype(jnp.int32)
    srcb2 = src_b.reshape(BE // IDX_W, IDX_W)
    dstb2 = dst_b.reshape(BE // IDX_W, IDX_W)

    for (Wm, bm, Wn, bn) in mp_params:
        in_dim = h.shape[1]
        P, Q = _proj2(h, Wm[:in_dim], bm, Wm[in_dim:])
        agg = _edge_agg(P, Q, srcb2, dstb2)
        h = _update(h, agg, Wn[:in_dim], bn, Wn[in_dim:])

    U, V = _proj2(h, edge_w1[:H], edge_b1, edge_w1[H:])
    mid = _silu(U[src_b] + V[dst_b])
    logits = (mid @ edge_w2 + edge_b2).squeeze(-1)
    rates = jax.nn.softplus(logits).reshape(B, E)

    rm = jnp.zeros((B, N, N), jnp.float32)
    rm = rm.at[:, src, dst].set(rates)
    diag = jnp.arange(N)
    row_sum = rm.sum(axis=-1)
    rm = rm.at[:, diag, diag].set(-row_sum)
    return rm
